# baseline (speedup 1.0000x reference)
import functools

import jax
import jax.numpy as jnp
from jax import lax
from jax.experimental import pallas as pl
from jax.experimental.pallas import tpu as pltpu
from jax.experimental.pallas import tpu_sc as plsc

BATCH = 16384
DIM = 64
DIM_MLP = 256
CHUNK = 128


def _make_gather(d, use_tc_tiling):
    info = plsc.get_sparse_core_info()
    nc, ns = info.num_cores, info.num_subcores
    nw = nc * ns
    b_per_w = BATCH // nw
    n_chunks = b_per_w // CHUNK
    mesh = plsc.VectorSubcoreMesh(core_axis_name="c", subcore_axis_name="s")

    f32 = jnp.float32

    @functools.partial(
        pl.kernel,
        mesh=mesh,
        out_type=[
            jax.ShapeDtypeStruct((BATCH, d), f32),
            jax.ShapeDtypeStruct((BATCH, d), f32),
        ],
        scratch_types=[
            pltpu.VMEM((n_chunks, CHUNK), jnp.int32),
            pltpu.VMEM((n_chunks, CHUNK), jnp.int32),
            pltpu.VMEM((CHUNK, d), f32),
            pltpu.VMEM((CHUNK, d), f32),
            pltpu.SemaphoreType.DMA,
            pltpu.SemaphoreType.DMA,
        ],
        compiler_params=pltpu.CompilerParams(use_tc_tiling_on_sc=use_tc_tiling),
    )
    def gather_kernel(u_idx_hbm, s_idx_hbm, ut_hbm, it_hbm,
                      out_u, out_i,
                      idx_u, idx_i, buf_a, buf_b, sem_a, sem_b):
        wid = lax.axis_index("s") * nc + lax.axis_index("c")
        base = wid * b_per_w
        for j in range(n_chunks):
            pltpu.sync_copy(u_idx_hbm.at[pl.ds(base + j * CHUNK, CHUNK)],
                            idx_u.at[j])
            pltpu.sync_copy(s_idx_hbm.at[pl.ds(base + j * CHUNK, CHUNK)],
                            idx_i.at[j])
        for j in range(n_chunks):
            row0 = base + j * CHUNK
            pltpu.async_copy(ut_hbm.at[idx_u.at[j]], buf_a, sem_a).wait()
            pltpu.sync_copy(buf_a, out_u.at[pl.ds(row0, CHUNK)])
            pltpu.async_copy(it_hbm.at[idx_i.at[j]], buf_b, sem_b).wait()
            pltpu.sync_copy(buf_b, out_i.at[pl.ds(row0, CHUNK)])

    return gather_kernel


_gather_mlp = _make_gather(DIM_MLP, True)


def _make_gather_gmf():
    info = plsc.get_sparse_core_info()
    nc, ns = info.num_cores, info.num_subcores
    nw = nc * ns
    b_per_w = BATCH // nw
    n_chunks = b_per_w // CHUNK
    mesh = plsc.VectorSubcoreMesh(core_axis_name="c", subcore_axis_name="s")
    f32 = jnp.float32
    i32 = jnp.int32
    L = 16

    @functools.partial(
        pl.kernel,
        mesh=mesh,
        out_type=[
            jax.ShapeDtypeStruct((BATCH, DIM), f32),
            jax.ShapeDtypeStruct((BATCH, DIM), f32),
        ],
        scratch_types=[
            pltpu.VMEM((n_chunks, CHUNK), i32),
            pltpu.VMEM((n_chunks, CHUNK), i32),
            pltpu.VMEM((CHUNK,), i32),
            pltpu.VMEM((CHUNK, 2 * DIM), f32),
            pltpu.VMEM((CHUNK, DIM), f32),
            pltpu.SemaphoreType.DMA,
        ],
        compiler_params=pltpu.CompilerParams(needs_layout_passes=False),
    )
    def gather_gmf(u_idx_hbm, s_idx_hbm, ut2_hbm, it2_hbm,
                   out_u, out_i,
                   idx_u, idx_i, idx2, buf2, sel, sem):
        wid = lax.axis_index("s") * nc + lax.axis_index("c")
        base = wid * b_per_w
        for j in range(n_chunks):
            pltpu.sync_copy(u_idx_hbm.at[pl.ds(base + j * CHUNK, CHUNK)],
                            idx_u.at[j])
            pltpu.sync_copy(s_idx_hbm.at[pl.ds(base + j * CHUNK, CHUNK)],
                            idx_i.at[j])

        def do_table(idx_ref, table_hbm, out_hbm, j):
            row0 = base + j * CHUNK
            for k in range(CHUNK // L):
                idx2[pl.ds(k * L, L)] = lax.shift_right_logical(
                    idx_ref[j, pl.ds(k * L, L)], 1)
            pltpu.async_copy(table_hbm.at[idx2], buf2, sem).wait()

            for g in range(CHUNK // L):
                rows = jnp.full((L,), g * L, dtype=i32) + lax.iota(i32, L)
                par = (idx_ref[j, pl.ds(g * L, L)] & 1) * DIM

                def body(c, _):
                    v = plsc.load_gather(buf2, [rows, par + c])
                    plsc.store_scatter(sel, [rows, jnp.full((L,), 0, i32) + c], v)
                    return _

                lax.fori_loop(0, DIM, body, 0)
            pltpu.sync_copy(sel, out_hbm.at[pl.ds(row0, CHUNK)])

        for j in range(n_chunks):
            do_table(idx_u, ut2_hbm, out_u, j)
            do_table(idx_i, it2_hbm, out_i, j)

    return gather_gmf


_gather_gmf = _make_gather_gmf()


def _mlp_body(ug, ig, um, im, w0u, w0i, b0, w1, b1, w2, b2, wpg, wpx, bp, out):
    f32 = jnp.float32
    x = jnp.dot(um[...], w0u[...], preferred_element_type=f32)
    x = x + jnp.dot(im[...], w0i[...], preferred_element_type=f32)
    x = jnp.maximum(x + b0[...], 0.0)
    x = jnp.maximum(jnp.dot(x, w1[...], preferred_element_type=f32) + b1[...], 0.0)
    x = jnp.maximum(jnp.dot(x, w2[...], preferred_element_type=f32) + b2[...], 0.0)
    g = ug[...] * ig[...]
    pred = (jnp.dot(g, wpg[...], preferred_element_type=f32)
            + jnp.dot(x, wpx[...], preferred_element_type=f32)
            + bp[...])
    out[...] = pred


def _run_mlp(ug, ig, um, im, W0, b0, W1, b1, W2, b2, Wp, bp):
    T = 1024
    grid = (BATCH // T,)
    f32 = jnp.float32
    w0u = W0[:, :DIM_MLP].T
    w0i = W0[:, DIM_MLP:].T
    w1 = W1.T
    w2 = W2.T
    wpg = Wp[:, :DIM].T
    wpx = Wp[:, DIM:].T
    b0r = b0.reshape(1, -1)
    b1r = b1.reshape(1, -1)
    b2r = b2.reshape(1, -1)
    bpr = bp.reshape(1, 1)

    batch_spec = lambda d: pl.BlockSpec((T, d), lambda i: (i, 0))
    full_spec = lambda a, b: pl.BlockSpec((a, b), lambda i: (0, 0))

    out = pl.pallas_call(
        _mlp_body,
        grid=grid,
        in_specs=[
            batch_spec(DIM), batch_spec(DIM), batch_spec(DIM_MLP), batch_spec(DIM_MLP),
            full_spec(256, 256), full_spec(256, 256), full_spec(1, 256),
            full_spec(256, 128), full_spec(1, 128),
            full_spec(128, 64), full_spec(1, 64),
            full_spec(64, 1), full_spec(64, 1), full_spec(1, 1),
        ],
        out_specs=pl.BlockSpec((T, 1), lambda i: (i, 0)),
        out_shape=jax.ShapeDtypeStruct((BATCH, 1), f32),
    )(ug, ig, um, im, w0u, w0i, b0r, w1, b1r, w2, b2r, wpg, wpx, bpr)
    return out.reshape(-1)


def kernel(userIdx, servIdx, U_gmf, U_mlp, I_gmf, I_mlp, W0, b0, W1, b1, W2, b2, Wp, bp):
    ui = userIdx.astype(jnp.int32)
    si = servIdx.astype(jnp.int32)
    um, im = _gather_mlp(ui, si, U_mlp, I_mlp)
    ug, ig = _gather_gmf(ui, si,
                         U_gmf.reshape(-1, 2 * DIM),
                         I_gmf.reshape(-1, 2 * DIM))
    return _run_mlp(ug, ig, um, im, W0, b0, W1, b1, W2, b2, Wp, bp)


# speedup vs baseline: 1.2300x; 1.2300x over previous
import functools

import jax
import jax.numpy as jnp
from jax import lax
from jax.experimental import pallas as pl
from jax.experimental.pallas import tpu as pltpu
from jax.experimental.pallas import tpu_sc as plsc

BATCH = 16384
DIM = 64
DIM_MLP = 256
CHUNK = 128


def _make_gather(d, use_tc_tiling):
    info = plsc.get_sparse_core_info()
    nc, ns = info.num_cores, info.num_subcores
    nw = nc * ns
    b_per_w = BATCH // nw
    n_chunks = b_per_w // CHUNK
    mesh = plsc.VectorSubcoreMesh(core_axis_name="c", subcore_axis_name="s")

    f32 = jnp.float32

    @functools.partial(
        pl.kernel,
        mesh=mesh,
        out_type=[
            jax.ShapeDtypeStruct((BATCH, d), f32),
            jax.ShapeDtypeStruct((BATCH, d), f32),
        ],
        scratch_types=[
            pltpu.VMEM((n_chunks, CHUNK), jnp.int32),
            pltpu.VMEM((n_chunks, CHUNK), jnp.int32),
            pltpu.VMEM((CHUNK, d), f32),
            pltpu.VMEM((CHUNK, d), f32),
            pltpu.SemaphoreType.DMA,
            pltpu.SemaphoreType.DMA,
        ],
        compiler_params=pltpu.CompilerParams(use_tc_tiling_on_sc=use_tc_tiling),
    )
    def gather_kernel(u_idx_hbm, s_idx_hbm, ut_hbm, it_hbm,
                      out_u, out_i,
                      idx_u, idx_i, buf_a, buf_b, sem_a, sem_b):
        wid = lax.axis_index("s") * nc + lax.axis_index("c")
        base = wid * b_per_w
        for j in range(n_chunks):
            pltpu.sync_copy(u_idx_hbm.at[pl.ds(base + j * CHUNK, CHUNK)],
                            idx_u.at[j])
            pltpu.sync_copy(s_idx_hbm.at[pl.ds(base + j * CHUNK, CHUNK)],
                            idx_i.at[j])
        for j in range(n_chunks):
            row0 = base + j * CHUNK
            pltpu.async_copy(ut_hbm.at[idx_u.at[j]], buf_a, sem_a).wait()
            pltpu.sync_copy(buf_a, out_u.at[pl.ds(row0, CHUNK)])
            pltpu.async_copy(it_hbm.at[idx_i.at[j]], buf_b, sem_b).wait()
            pltpu.sync_copy(buf_b, out_i.at[pl.ds(row0, CHUNK)])

    return gather_kernel


_gather_mlp = _make_gather(DIM_MLP, True)


_gather_gmf = _make_gather(DIM, False)


def _mlp_body(ug, ig, um, im, w0u, w0i, b0, w1, b1, w2, b2, wpg, wpx, bp, out):
    f32 = jnp.float32
    bf16 = jnp.bfloat16

    def bdot(a, b):
        return jnp.dot(a.astype(bf16), b.astype(bf16), preferred_element_type=f32)

    x = bdot(um[...], w0u[...]) + bdot(im[...], w0i[...])
    x = jnp.maximum(x + b0[...], 0.0)
    x = jnp.maximum(bdot(x, w1[...]) + b1[...], 0.0)
    x = jnp.maximum(bdot(x, w2[...]) + b2[...], 0.0)
    g = ug[...] * ig[...]
    pred = bdot(g, wpg[...]) + bdot(x, wpx[...]) + bp[...]
    out[...] = pred


def _run_mlp(ug, ig, um, im, W0, b0, W1, b1, W2, b2, Wp, bp):
    T = 1024
    grid = (BATCH // T,)
    f32 = jnp.float32
    w0u = W0[:, :DIM_MLP].T
    w0i = W0[:, DIM_MLP:].T
    w1 = W1.T
    w2 = W2.T
    wpg = Wp[:, :DIM].T
    wpx = Wp[:, DIM:].T
    b0r = b0.reshape(1, -1)
    b1r = b1.reshape(1, -1)
    b2r = b2.reshape(1, -1)
    bpr = bp.reshape(1, 1)

    batch_spec = lambda d: pl.BlockSpec((T, d), lambda i: (i, 0))
    full_spec = lambda a, b: pl.BlockSpec((a, b), lambda i: (0, 0))

    out = pl.pallas_call(
        _mlp_body,
        grid=grid,
        in_specs=[
            batch_spec(DIM), batch_spec(DIM), batch_spec(DIM_MLP), batch_spec(DIM_MLP),
            full_spec(256, 256), full_spec(256, 256), full_spec(1, 256),
            full_spec(256, 128), full_spec(1, 128),
            full_spec(128, 64), full_spec(1, 64),
            full_spec(64, 1), full_spec(64, 1), full_spec(1, 1),
        ],
        out_specs=pl.BlockSpec((T, 1), lambda i: (i, 0)),
        out_shape=jax.ShapeDtypeStruct((BATCH, 1), f32),
    )(ug, ig, um, im, w0u, w0i, b0r, w1, b1r, w2, b2r, wpg, wpx, bpr)
    return out.reshape(-1)


def kernel(userIdx, servIdx, U_gmf, U_mlp, I_gmf, I_mlp, W0, b0, W1, b1, W2, b2, Wp, bp):
    ui = userIdx.astype(jnp.int32)
    si = servIdx.astype(jnp.int32)
    um, im = _gather_mlp(ui, si, U_mlp, I_mlp)
    ug, ig = _gather_gmf(ui, si, U_gmf, I_gmf)
    return _run_mlp(ug, ig, um, im, W0, b0, W1, b1, W2, b2, Wp, bp)


# speedup vs baseline: 1.3026x; 1.0590x over previous
import functools

import jax
import jax.numpy as jnp
from jax import lax
from jax.experimental import pallas as pl
from jax.experimental.pallas import tpu as pltpu
from jax.experimental.pallas import tpu_sc as plsc

BATCH = 16384
DIM = 64
DIM_MLP = 256
CHUNK = 128


def _make_gather_all():
    info = plsc.get_sparse_core_info()
    nc, ns = info.num_cores, info.num_subcores
    nw = nc * ns
    b_per_w = BATCH // nw
    n_chunks = b_per_w // CHUNK
    mesh = plsc.VectorSubcoreMesh(core_axis_name="c", subcore_axis_name="s")
    f32 = jnp.float32

    @functools.partial(
        pl.kernel,
        mesh=mesh,
        out_type=[
            jax.ShapeDtypeStruct((BATCH, DIM_MLP), f32),
            jax.ShapeDtypeStruct((BATCH, DIM_MLP), f32),
            jax.ShapeDtypeStruct((BATCH, 2 * DIM), f32),
            jax.ShapeDtypeStruct((BATCH, 2 * DIM), f32),
        ],
        scratch_types=[
            pltpu.VMEM((n_chunks, CHUNK), jnp.int32),
            pltpu.VMEM((n_chunks, CHUNK), jnp.int32),
            pltpu.VMEM((CHUNK, DIM_MLP), f32),
            pltpu.VMEM((CHUNK, DIM_MLP), f32),
            pltpu.VMEM((CHUNK, 2 * DIM), f32),
            pltpu.VMEM((CHUNK, 2 * DIM), f32),
            pltpu.SemaphoreType.DMA,
            pltpu.SemaphoreType.DMA,
        ],
    )
    def gather_all(u_idx_hbm, s_idx_hbm, um_hbm, im_hbm, ugp_hbm, igp_hbm,
                   out_um, out_im, out_ug, out_ig,
                   idx_u, idx_i, buf_ma, buf_mb, buf_ga, buf_gb,
                   sem_a, sem_b):
        wid = lax.axis_index("s") * nc + lax.axis_index("c")
        base = wid * b_per_w
        for j in range(n_chunks):
            pltpu.sync_copy(u_idx_hbm.at[pl.ds(base + j * CHUNK, CHUNK)],
                            idx_u.at[j])
            pltpu.sync_copy(s_idx_hbm.at[pl.ds(base + j * CHUNK, CHUNK)],
                            idx_i.at[j])
        for j in range(n_chunks):
            row0 = base + j * CHUNK
            sl = pl.ds(row0, CHUNK)
            pltpu.async_copy(um_hbm.at[idx_u.at[j]], buf_ma, sem_a).wait()
            pltpu.sync_copy(buf_ma, out_um.at[sl])
            pltpu.async_copy(im_hbm.at[idx_i.at[j]], buf_mb, sem_b).wait()
            pltpu.sync_copy(buf_mb, out_im.at[sl])
            pltpu.async_copy(ugp_hbm.at[idx_u.at[j]], buf_ga, sem_a).wait()
            pltpu.sync_copy(buf_ga, out_ug.at[sl])
            pltpu.async_copy(igp_hbm.at[idx_i.at[j]], buf_gb, sem_b).wait()
            pltpu.sync_copy(buf_gb, out_ig.at[sl])

    return gather_all


_gather_all = _make_gather_all()


def _mlp_body(ug, ig, um, im, w0u, w0i, b0, w1, b1, w2, b2, wpg, wpx, bp, out):
    f32 = jnp.float32
    bf16 = jnp.bfloat16

    def bdot(a, b):
        return jnp.dot(a.astype(bf16), b.astype(bf16), preferred_element_type=f32)

    x = bdot(um[...], w0u[...]) + bdot(im[...], w0i[...])
    x = jnp.maximum(x + b0[...], 0.0)
    x = jnp.maximum(bdot(x, w1[...]) + b1[...], 0.0)
    x = jnp.maximum(bdot(x, w2[...]) + b2[...], 0.0)
    g = ug[:, :DIM] * ig[:, :DIM]
    pred = bdot(g, wpg[...]) + bdot(x, wpx[...]) + bp[...]
    out[...] = pred


def _run_mlp(ug2, ig2, um, im, W0, b0, W1, b1, W2, b2, Wp, bp):
    T = 1024
    grid = (BATCH // T,)
    f32 = jnp.float32
    w0u = W0[:, :DIM_MLP].T
    w0i = W0[:, DIM_MLP:].T
    w1 = W1.T
    w2 = W2.T
    wpg = Wp[:, :DIM].T
    wpx = Wp[:, DIM:].T
    b0r = b0.reshape(1, -1)
    b1r = b1.reshape(1, -1)
    b2r = b2.reshape(1, -1)
    bpr = bp.reshape(1, 1)

    batch_spec = lambda d: pl.BlockSpec((T, d), lambda i: (i, 0))
    full_spec = lambda a, b: pl.BlockSpec((a, b), lambda i: (0, 0))

    out = pl.pallas_call(
        _mlp_body,
        grid=grid,
        in_specs=[
            batch_spec(2 * DIM), batch_spec(2 * DIM), batch_spec(DIM_MLP), batch_spec(DIM_MLP),
            full_spec(256, 256), full_spec(256, 256), full_spec(1, 256),
            full_spec(256, 128), full_spec(1, 128),
            full_spec(128, 64), full_spec(1, 64),
            full_spec(64, 1), full_spec(64, 1), full_spec(1, 1),
        ],
        out_specs=pl.BlockSpec((T, 1), lambda i: (i, 0)),
        out_shape=jax.ShapeDtypeStruct((BATCH, 1), f32),
    )(ug2, ig2, um, im, w0u, w0i, b0r, w1, b1r, w2, b2r, wpg, wpx, bpr)
    return out.reshape(-1)


def kernel(userIdx, servIdx, U_gmf, U_mlp, I_gmf, I_mlp, W0, b0, W1, b1, W2, b2, Wp, bp):
    ui = userIdx.astype(jnp.int32)
    si = servIdx.astype(jnp.int32)
    ugp = jnp.pad(U_gmf, ((0, 0), (0, DIM)))
    igp = jnp.pad(I_gmf, ((0, 0), (0, DIM)))
    um, im, ug2, ig2 = _gather_all(ui, si, U_mlp, I_mlp, ugp, igp)
    return _run_mlp(ug2, ig2, um, im, W0, b0, W1, b1, W2, b2, Wp, bp)


# speedup vs baseline: 1.3237x; 1.0162x over previous
import functools

import jax
import jax.numpy as jnp
from jax import lax
from jax.experimental import pallas as pl
from jax.experimental.pallas import tpu as pltpu
from jax.experimental.pallas import tpu_sc as plsc

BATCH = 16384
DIM = 64
DIM_MLP = 256
CHUNK = 64


def _make_gather(d):
    info = plsc.get_sparse_core_info()
    nc, ns = info.num_cores, info.num_subcores
    nw = nc * ns
    b_per_w = BATCH // nw
    n_chunks = b_per_w // CHUNK
    mesh = plsc.VectorSubcoreMesh(core_axis_name="c", subcore_axis_name="s")
    f32 = jnp.float32

    @functools.partial(
        pl.kernel,
        mesh=mesh,
        out_type=[
            jax.ShapeDtypeStruct((BATCH, d), f32),
            jax.ShapeDtypeStruct((BATCH, d), f32),
        ],
        scratch_types=[
            pltpu.VMEM((n_chunks, CHUNK), jnp.int32),
            pltpu.VMEM((n_chunks, CHUNK), jnp.int32),
            pltpu.VMEM((CHUNK, d), f32),
            pltpu.VMEM((CHUNK, d), f32),
            pltpu.VMEM((CHUNK, d), f32),
            pltpu.VMEM((CHUNK, d), f32),
            pltpu.SemaphoreType.DMA,
            pltpu.SemaphoreType.DMA,
            pltpu.SemaphoreType.DMA,
            pltpu.SemaphoreType.DMA,
        ],
    )
    def gather_kernel(u_idx_hbm, s_idx_hbm, ut_hbm, it_hbm,
                      out_u, out_i,
                      idx_u, idx_i, bu0, bu1, bi0, bi1,
                      su0, su1, si0, si1):
        wid = lax.axis_index("s") * nc + lax.axis_index("c")
        base = wid * b_per_w
        bufs_u, bufs_i = (bu0, bu1), (bi0, bi1)
        sems_u, sems_i = (su0, su1), (si0, si1)
        for j in range(n_chunks):
            pltpu.sync_copy(u_idx_hbm.at[pl.ds(base + j * CHUNK, CHUNK)],
                            idx_u.at[j])
            pltpu.sync_copy(s_idx_hbm.at[pl.ds(base + j * CHUNK, CHUNK)],
                            idx_i.at[j])

        cps = [None, None]
        cps[0] = (
            pltpu.async_copy(ut_hbm.at[idx_u.at[0]], bufs_u[0], sems_u[0]),
            pltpu.async_copy(it_hbm.at[idx_i.at[0]], bufs_i[0], sems_i[0]),
        )
        for j in range(n_chunks):
            s = j % 2
            n = (j + 1) % 2
            if j + 1 < n_chunks:
                cps[n] = (
                    pltpu.async_copy(ut_hbm.at[idx_u.at[j + 1]],
                                     bufs_u[n], sems_u[n]),
                    pltpu.async_copy(it_hbm.at[idx_i.at[j + 1]],
                                     bufs_i[n], sems_i[n]),
                )
            sl = pl.ds(base + j * CHUNK, CHUNK)
            cps[s][0].wait()
            pltpu.sync_copy(bufs_u[s], out_u.at[sl])
            cps[s][1].wait()
            pltpu.sync_copy(bufs_i[s], out_i.at[sl])

    return gather_kernel


_gather_mlp = _make_gather(DIM_MLP)
_gather_gmf = _make_gather(2 * DIM)


def _mlp_body(ug, ig, um, im, w0u, w0i, b0, w1, b1, w2, b2, wpg, wpx, bp, out):
    f32 = jnp.float32
    bf16 = jnp.bfloat16

    def bdot(a, b):
        return jnp.dot(a.astype(bf16), b.astype(bf16), preferred_element_type=f32)

    x = bdot(um[...], w0u[...]) + bdot(im[...], w0i[...])
    x = jnp.maximum(x + b0[...], 0.0)
    x = jnp.maximum(bdot(x, w1[...]) + b1[...], 0.0)
    x = jnp.maximum(bdot(x, w2[...]) + b2[...], 0.0)
    g = ug[:, :DIM] * ig[:, :DIM]
    pred = bdot(g, wpg[...]) + bdot(x, wpx[...]) + bp[...]
    out[...] = pred


def _run_mlp(ug2, ig2, um, im, W0, b0, W1, b1, W2, b2, Wp, bp):
    T = 1024
    grid = (BATCH // T,)
    f32 = jnp.float32
    w0u = W0[:, :DIM_MLP].T
    w0i = W0[:, DIM_MLP:].T
    w1 = W1.T
    w2 = W2.T
    wpg = Wp[:, :DIM].T
    wpx = Wp[:, DIM:].T
    b0r = b0.reshape(1, -1)
    b1r = b1.reshape(1, -1)
    b2r = b2.reshape(1, -1)
    bpr = bp.reshape(1, 1)

    batch_spec = lambda d: pl.BlockSpec((T, d), lambda i: (i, 0))
    full_spec = lambda a, b: pl.BlockSpec((a, b), lambda i: (0, 0))

    out = pl.pallas_call(
        _mlp_body,
        grid=grid,
        in_specs=[
            batch_spec(2 * DIM), batch_spec(2 * DIM),
            batch_spec(DIM_MLP), batch_spec(DIM_MLP),
            full_spec(256, 256), full_spec(256, 256), full_spec(1, 256),
            full_spec(256, 128), full_spec(1, 128),
            full_spec(128, 64), full_spec(1, 64),
            full_spec(64, 1), full_spec(64, 1), full_spec(1, 1),
        ],
        out_specs=pl.BlockSpec((T, 1), lambda i: (i, 0)),
        out_shape=jax.ShapeDtypeStruct((BATCH, 1), f32),
    )(ug2, ig2, um, im, w0u, w0i, b0r, w1, b1r, w2, b2r, wpg, wpx, bpr)
    return out.reshape(-1)


def kernel(userIdx, servIdx, U_gmf, U_mlp, I_gmf, I_mlp, W0, b0, W1, b1, W2, b2, Wp, bp):
    ui = userIdx.astype(jnp.int32)
    si = servIdx.astype(jnp.int32)
    um, im = _gather_mlp(ui, si, U_mlp, I_mlp)
    ugp = jnp.pad(U_gmf, ((0, 0), (0, DIM)))
    igp = jnp.pad(I_gmf, ((0, 0), (0, DIM)))
    ug2, ig2 = _gather_gmf(ui, si, ugp, igp)
    return _run_mlp(ug2, ig2, um, im, W0, b0, W1, b1, W2, b2, Wp, bp)


# speedup vs baseline: 1.3289x; 1.0040x over previous
import functools

import jax
import jax.numpy as jnp
from jax import lax
from jax.experimental import pallas as pl
from jax.experimental.pallas import tpu as pltpu
from jax.experimental.pallas import tpu_sc as plsc

BATCH = 16384
DIM = 64
DIM_MLP = 256
CHUNK = 64


def _make_gather(d):
    info = plsc.get_sparse_core_info()
    nc, ns = info.num_cores, info.num_subcores
    nw = nc * ns
    b_per_w = BATCH // nw
    n_chunks = b_per_w // CHUNK
    mesh = plsc.VectorSubcoreMesh(core_axis_name="c", subcore_axis_name="s")
    f32 = jnp.float32

    @functools.partial(
        pl.kernel,
        mesh=mesh,
        out_type=[
            jax.ShapeDtypeStruct((BATCH, d), f32),
            jax.ShapeDtypeStruct((BATCH, d), f32),
        ],
        scratch_types=[
            pltpu.VMEM((n_chunks, CHUNK), jnp.int32),
            pltpu.VMEM((n_chunks, CHUNK), jnp.int32),
            pltpu.VMEM((CHUNK, d), f32),
            pltpu.VMEM((CHUNK, d), f32),
            pltpu.VMEM((CHUNK, d), f32),
            pltpu.VMEM((CHUNK, d), f32),
            pltpu.SemaphoreType.DMA,
            pltpu.SemaphoreType.DMA,
            pltpu.SemaphoreType.DMA,
            pltpu.SemaphoreType.DMA,
        ],
    )
    def gather_kernel(u_idx_hbm, s_idx_hbm, ut_hbm, it_hbm,
                      out_u, out_i,
                      idx_u, idx_i, bu0, bu1, bi0, bi1,
                      su0, su1, si0, si1):
        wid = lax.axis_index("s") * nc + lax.axis_index("c")
        base = wid * b_per_w
        bufs_u, bufs_i = (bu0, bu1), (bi0, bi1)
        sems_u, sems_i = (su0, su1), (si0, si1)
        for j in range(n_chunks):
            pltpu.sync_copy(u_idx_hbm.at[pl.ds(base + j * CHUNK, CHUNK)],
                            idx_u.at[j])
            pltpu.sync_copy(s_idx_hbm.at[pl.ds(base + j * CHUNK, CHUNK)],
                            idx_i.at[j])

        cps = [None, None]
        cps[0] = (
            pltpu.async_copy(ut_hbm.at[idx_u.at[0]], bufs_u[0], sems_u[0]),
            pltpu.async_copy(it_hbm.at[idx_i.at[0]], bufs_i[0], sems_i[0]),
        )
        for j in range(n_chunks):
            s = j % 2
            n = (j + 1) % 2
            if j + 1 < n_chunks:
                cps[n] = (
                    pltpu.async_copy(ut_hbm.at[idx_u.at[j + 1]],
                                     bufs_u[n], sems_u[n]),
                    pltpu.async_copy(it_hbm.at[idx_i.at[j + 1]],
                                     bufs_i[n], sems_i[n]),
                )
            sl = pl.ds(base + j * CHUNK, CHUNK)
            cps[s][0].wait()
            pltpu.sync_copy(bufs_u[s], out_u.at[sl])
            cps[s][1].wait()
            pltpu.sync_copy(bufs_i[s], out_i.at[sl])

    return gather_kernel


_gather_mlp = _make_gather(DIM_MLP)
_gather_gmf = _make_gather(2 * DIM)


def _bdot(a, b):
    return jnp.dot(a.astype(jnp.bfloat16), b.astype(jnp.bfloat16),
                   preferred_element_type=jnp.float32)


def _dense_body(um, im, w0u, w0i, b0, w1, b1, w2, b2, wpx, bp, out):
    x = _bdot(um[...], w0u[...]) + _bdot(im[...], w0i[...])
    x = jnp.maximum(x + b0[...], 0.0)
    x = jnp.maximum(_bdot(x, w1[...]) + b1[...], 0.0)
    x = jnp.maximum(_bdot(x, w2[...]) + b2[...], 0.0)
    out[...] = _bdot(x, wpx[...]) + bp[...]


def _final_body(ug, ig, xd, wpg, out):
    g = ug[:, :DIM] * ig[:, :DIM]
    out[...] = _bdot(g, wpg[...]) + xd[...]


_T = 1024


def _run_dense(um, im, W0, b0, W1, b1, W2, b2, Wp, bp):
    grid = (BATCH // _T,)
    f32 = jnp.float32
    w0u = W0[:, :DIM_MLP].T
    w0i = W0[:, DIM_MLP:].T
    w1 = W1.T
    w2 = W2.T
    wpx = Wp[:, DIM:].T
    b0r = b0.reshape(1, -1)
    b1r = b1.reshape(1, -1)
    b2r = b2.reshape(1, -1)
    bpr = bp.reshape(1, 1)

    batch_spec = lambda d: pl.BlockSpec((_T, d), lambda i: (i, 0))
    full_spec = lambda a, b: pl.BlockSpec((a, b), lambda i: (0, 0))

    return pl.pallas_call(
        _dense_body,
        grid=grid,
        in_specs=[
            batch_spec(DIM_MLP), batch_spec(DIM_MLP),
            full_spec(256, 256), full_spec(256, 256), full_spec(1, 256),
            full_spec(256, 128), full_spec(1, 128),
            full_spec(128, 64), full_spec(1, 64),
            full_spec(64, 1), full_spec(1, 1),
        ],
        out_specs=pl.BlockSpec((_T, 1), lambda i: (i, 0)),
        out_shape=jax.ShapeDtypeStruct((BATCH, 1), f32),
    )(um, im, w0u, w0i, b0r, w1, b1r, w2, b2r, wpx, bpr)


def _run_final(ug2, ig2, xdot, Wp):
    grid = (BATCH // _T,)
    f32 = jnp.float32
    wpg = Wp[:, :DIM].T
    batch_spec = lambda d: pl.BlockSpec((_T, d), lambda i: (i, 0))
    out = pl.pallas_call(
        _final_body,
        grid=grid,
        in_specs=[
            batch_spec(2 * DIM), batch_spec(2 * DIM), batch_spec(1),
            pl.BlockSpec((DIM, 1), lambda i: (0, 0)),
        ],
        out_specs=pl.BlockSpec((_T, 1), lambda i: (i, 0)),
        out_shape=jax.ShapeDtypeStruct((BATCH, 1), f32),
    )(ug2, ig2, xdot, wpg)
    return out.reshape(-1)


def kernel(userIdx, servIdx, U_gmf, U_mlp, I_gmf, I_mlp, W0, b0, W1, b1, W2, b2, Wp, bp):
    ui = userIdx.astype(jnp.int32)
    si = servIdx.astype(jnp.int32)
    um, im = _gather_mlp(ui, si, U_mlp, I_mlp)
    ugp = jnp.pad(U_gmf, ((0, 0), (0, DIM)))
    igp = jnp.pad(I_gmf, ((0, 0), (0, DIM)))
    ug2, ig2 = _gather_gmf(ui, si, ugp, igp)
    xdot = _run_dense(um, im, W0, b0, W1, b1, W2, b2, Wp, bp)
    return _run_final(ug2, ig2, xdot, Wp)
